# baseline (speedup 1.0000x reference)
import jax
import jax.numpy as jnp
from jax import lax
from jax.experimental import pallas as pl
from jax.experimental.pallas import tpu as pltpu
from jax.experimental.pallas import tpu_sc as plsc

_N = 100000
_V = 120
_D = 256
_NW = 32
_CH = 112
_NB = 2
_NCH = 28
_BPW = _CH * _NCH
_G = _CH // 16


def _embed_body(idx_hbm, table_hbm, out_hbm, idx_v, table_v, buf0, buf1,
                wsem0, wsem1):
    bufs = (buf0, buf1)
    wsems = (wsem0, wsem1)
    wid = lax.axis_index("s") * 2 + lax.axis_index("c")
    base = jnp.minimum(wid * _BPW, _N - _BPW)

    pltpu.sync_copy(table_hbm, table_v)
    pltpu.sync_copy(idx_hbm.at[pl.ds(base, _BPW)], idx_v)

    lanes16 = lax.iota(jnp.int32, 16)
    drows = [lanes16 + g * 16 for g in range(_G)]

    def assemble(c, b):
        srcs = [
            (idx_v[pl.ds(c * _CH + g * 16, 16)] - 1) * _D for g in range(_G)
        ]

        @plsc.parallel_loop(0, _D, unroll=8)
        def jbody(j):
            col = (j + lanes16) & (_D - 1)
            for g in range(_G):
                v = plsc.load_gather(table_v, [srcs[g] + col])
                plsc.store_scatter(bufs[b], [drows[g], col], v)

    def start_write(c, b):
        pltpu.make_async_copy(
            bufs[b], out_hbm.at[pl.ds(base + c * _CH, _CH)], wsems[b]
        ).start()

    def wait_write(b):
        pltpu.make_async_copy(
            bufs[b], out_hbm.at[pl.ds(base, _CH)], wsems[b]
        ).wait()

    for b in range(_NB):
        assemble(b, b)
        start_write(b, b)

    def body(c2, carry):
        for b in range(_NB):
            c = c2 * _NB + b
            wait_write(b)
            assemble(c, b)
            start_write(c, b)
        return carry

    lax.fori_loop(1, _NCH // _NB, body, 0)

    for b in range(_NB):
        wait_write(b)


@jax.jit
def _embed_lookup(idx, table_flat):
    mesh = plsc.VectorSubcoreMesh(core_axis_name="c", subcore_axis_name="s")
    fn = pl.kernel(
        _embed_body,
        mesh=mesh,
        compiler_params=pltpu.CompilerParams(needs_layout_passes=False),
        out_type=jax.ShapeDtypeStruct((_N, _D), jnp.float32),
        scratch_types=(
            [pltpu.VMEM((_BPW,), jnp.int32),
             pltpu.VMEM((_V * _D,), jnp.float32)]
            + [pltpu.VMEM((_CH, _D), jnp.float32) for _ in range(_NB)]
            + [pltpu.SemaphoreType.DMA for _ in range(_NB)]
        ),
    )
    return fn(idx, table_flat)


def kernel(atomic_numbers, atom_embedding_weight):
    return _embed_lookup(atomic_numbers, atom_embedding_weight.reshape(-1))


# speedup vs baseline: 1.0983x; 1.0983x over previous
import jax
import jax.numpy as jnp
from jax import lax
from jax.experimental import pallas as pl
from jax.experimental.pallas import tpu as pltpu
from jax.experimental.pallas import tpu_sc as plsc

_N = 100000
_V = 120
_D = 256
_NW = 32
_CH = 112
_NB = 2
_NCH = 28
_BPW = _CH * _NCH
_G = _CH // 16


def _embed_body(idx_hbm, table_hbm, out_hbm, idx_v, table_v, buf0, buf1,
                wsem0, wsem1):
    bufs = (buf0, buf1)
    wsems = (wsem0, wsem1)
    wid = lax.axis_index("s") * 2 + lax.axis_index("c")
    base = jnp.minimum(wid * _BPW, _N - _BPW)

    pltpu.sync_copy(table_hbm, table_v)
    pltpu.sync_copy(idx_hbm.at[pl.ds(base, _BPW)], idx_v)

    lanes16 = lax.iota(jnp.int32, 16)
    drows = [lanes16 + g * 16 for g in range(_G)]

    def assemble(c, b):
        srcs = [
            (idx_v[pl.ds(c * _CH + g * 16, 16)] - 1) * _D for g in range(_G)
        ]

        @plsc.parallel_loop(0, _D, unroll=4)
        def jbody(j):
            col = (j + lanes16) & (_D - 1)
            for g in range(_G):
                v = plsc.load_gather(table_v, [srcs[g] + col])
                plsc.store_scatter(bufs[b], [drows[g], col], v)

    def start_write(c, b):
        pltpu.make_async_copy(
            bufs[b], out_hbm.at[pl.ds(base + c * _CH, _CH)], wsems[b]
        ).start()

    def wait_write(b):
        pltpu.make_async_copy(
            bufs[b], out_hbm.at[pl.ds(base, _CH)], wsems[b]
        ).wait()

    for b in range(_NB):
        assemble(b, b)
        start_write(b, b)

    def body(c2, carry):
        for b in range(_NB):
            c = c2 * _NB + b
            wait_write(b)
            assemble(c, b)
            start_write(c, b)
        return carry

    lax.fori_loop(1, _NCH // _NB, body, 0)

    for b in range(_NB):
        wait_write(b)


@jax.jit
def _embed_lookup(idx, table_flat):
    mesh = plsc.VectorSubcoreMesh(core_axis_name="c", subcore_axis_name="s")
    fn = pl.kernel(
        _embed_body,
        mesh=mesh,
        compiler_params=pltpu.CompilerParams(needs_layout_passes=False),
        out_type=jax.ShapeDtypeStruct((_N, _D), jnp.float32),
        scratch_types=(
            [pltpu.VMEM((_BPW,), jnp.int32),
             pltpu.VMEM((_V * _D,), jnp.float32)]
            + [pltpu.VMEM((_CH, _D), jnp.float32) for _ in range(_NB)]
            + [pltpu.SemaphoreType.DMA for _ in range(_NB)]
        ),
    )
    return fn(idx, table_flat)


def kernel(atomic_numbers, atom_embedding_weight):
    return _embed_lookup(atomic_numbers, atom_embedding_weight.reshape(-1))


# speedup vs baseline: 1.1012x; 1.0027x over previous
import jax
import jax.numpy as jnp
from jax import lax
from jax.experimental import pallas as pl
from jax.experimental.pallas import tpu as pltpu
from jax.experimental.pallas import tpu_sc as plsc

_N = 100000
_V = 120
_D = 256
_NW = 32
_CH = 96
_NB = 3
_NCH = 33
_BPW = _CH * _NCH
_G = _CH // 16


def _embed_body(idx_hbm, table_hbm, out_hbm, idx_v, table_v, buf0, buf1,
                buf2, wsem0, wsem1, wsem2):
    bufs = (buf0, buf1, buf2)
    wsems = (wsem0, wsem1, wsem2)
    wid = lax.axis_index("s") * 2 + lax.axis_index("c")
    base = jnp.minimum(wid * _BPW, _N - _BPW)

    pltpu.sync_copy(table_hbm, table_v)
    pltpu.sync_copy(idx_hbm.at[pl.ds(base, _BPW)], idx_v)

    lanes16 = lax.iota(jnp.int32, 16)
    drows = [lanes16 + g * 16 for g in range(_G)]

    def assemble(c, b):
        srcs = [
            (idx_v[pl.ds(c * _CH + g * 16, 16)] - 1) * _D for g in range(_G)
        ]

        @plsc.parallel_loop(0, _D, unroll=4)
        def jbody(j):
            col = (j + lanes16) & (_D - 1)
            for g in range(_G):
                v = plsc.load_gather(table_v, [srcs[g] + col])
                plsc.store_scatter(bufs[b], [drows[g], col], v)

    def start_write(c, b):
        pltpu.make_async_copy(
            bufs[b], out_hbm.at[pl.ds(base + c * _CH, _CH)], wsems[b]
        ).start()

    def wait_write(b):
        pltpu.make_async_copy(
            bufs[b], out_hbm.at[pl.ds(base, _CH)], wsems[b]
        ).wait()

    for b in range(_NB):
        assemble(b, b)
        start_write(b, b)

    def body(c2, carry):
        for b in range(_NB):
            c = c2 * _NB + b
            wait_write(b)
            assemble(c, b)
            start_write(c, b)
        return carry

    lax.fori_loop(1, _NCH // _NB, body, 0)

    for b in range(_NB):
        wait_write(b)


@jax.jit
def _embed_lookup(idx, table_flat):
    mesh = plsc.VectorSubcoreMesh(core_axis_name="c", subcore_axis_name="s")
    fn = pl.kernel(
        _embed_body,
        mesh=mesh,
        compiler_params=pltpu.CompilerParams(needs_layout_passes=False),
        out_type=jax.ShapeDtypeStruct((_N, _D), jnp.float32),
        scratch_types=(
            [pltpu.VMEM((_BPW,), jnp.int32),
             pltpu.VMEM((_V * _D,), jnp.float32)]
            + [pltpu.VMEM((_CH, _D), jnp.float32) for _ in range(_NB)]
            + [pltpu.SemaphoreType.DMA for _ in range(_NB)]
        ),
    )
    return fn(idx, table_flat)


def kernel(atomic_numbers, atom_embedding_weight):
    return _embed_lookup(atomic_numbers, atom_embedding_weight.reshape(-1))
